# baseline (speedup 1.0000x reference)
import functools

import jax
import jax.numpy as jnp
from jax import lax
from jax.experimental import pallas as pl
from jax.experimental.pallas import tpu as pltpu
from jax.experimental.pallas import tpu_sc as plsc

H = 512
W = 512
C = 96
NV = C // 16
SMAXP = 288
CP = C + 1
ROWCAP = SMAXP * CP
ROWF = W * C
NC, NS = 2, 16
NW = NC * NS
RPW = H // NW
XCH = 4
XCHW = ROWF // XCH

_mesh = plsc.VectorSubcoreMesh(
    core_axis_name="c", subcore_axis_name="s", num_cores=NC, num_subcores=NS
)


def _wid():
    return lax.axis_index("s") * NC + lax.axis_index("c")


def _sload(ref, i):
    return plsc.load_gather(ref, [jnp.full((16,), i, jnp.int32)])[0]


def _to_smem(vmem_ref, smem_ref, n):
    def gbody(g, _):
        v = vmem_ref[pl.ds(g * 16, 16)]
        for l in range(16):
            smem_ref[g * 16 + l] = v[l]
        return 0

    lax.fori_loop(0, n // 16, gbody, 0)


def _stage0_body(hm_r_ref, hm_c_ref, vm_r_ref, vm_c_ref,
                 rowid_ref, colid96_ref, cinv_ref, segpack_ref):
    ii_r = lax.broadcasted_iota(jnp.int32, (1, W), 1)
    ii_c = lax.broadcasted_iota(jnp.int32, (H, 1), 0)
    i0m = lax.broadcasted_iota(jnp.int32, (H, W), 0)
    i1m = lax.broadcasted_iota(jnp.int32, (H, W), 1)
    le = (i0m <= i1m).astype(jnp.float32)
    ge = (i0m >= i1m).astype(jnp.float32)

    def rise_row(m):
        prev = jnp.concatenate([jnp.zeros((1, 1), jnp.int32), m[:, : W - 1]], axis=1)
        return jnp.where((m - prev > 0) & (ii_r > 0), 1.0, 0.0).astype(jnp.float32)

    def rise_col(m):
        prev = jnp.concatenate([jnp.zeros((1, 1), jnp.int32), m[: H - 1, :]], axis=0)
        return jnp.where((m - prev > 0) & (ii_c > 0), 1.0, 0.0).astype(jnp.float32)

    hrise_r = rise_row(hm_r_ref[...])
    hrise_c = rise_col(hm_c_ref[...])
    vrise_r = rise_row(vm_r_ref[...])
    vrise_c = rise_col(vm_c_ref[...])

    rowid_r = jnp.dot(hrise_r, le, preferred_element_type=jnp.float32)
    rowid_c = jnp.dot(ge, hrise_c, preferred_element_type=jnp.float32)
    colid_r = jnp.dot(vrise_r, le, preferred_element_type=jnp.float32)
    colid_c = jnp.dot(ge, vrise_c, preferred_element_type=jnp.float32)

    ecol = (colid_c == colid_r).astype(jnp.float32)
    ncol = jnp.sum(ecol, axis=0, keepdims=True)

    cmp = (i0m == rowid_r.astype(jnp.int32)).astype(jnp.float32)
    is_start = jnp.where(ii_r == 0, 1.0, hrise_r)
    i_f = ii_r.astype(jnp.float32)
    segn = jnp.sum(cmp, axis=1, keepdims=True)
    segstart = jnp.sum(cmp * is_start * i_f, axis=1, keepdims=True)

    rowid_ref[...] = rowid_c.astype(jnp.int32)
    colid96_ref[...] = (colid_r * 97.0).astype(jnp.int32)
    cinv_ref[...] = 1.0 / ncol
    segpack_ref[...] = (segstart + segn * 1024.0).astype(jnp.int32)


_stage0 = pl.pallas_call(
    _stage0_body,
    out_shape=[
        jax.ShapeDtypeStruct((H, 1), jnp.int32),
        jax.ShapeDtypeStruct((1, W), jnp.int32),
        jax.ShapeDtypeStruct((1, W), jnp.float32),
        jax.ShapeDtypeStruct((H, 1), jnp.int32),
    ],
)


WCH = W // XCH


def _main_body(x_hbm, coloff_hbm, cinv_hbm, rowid_hbm, segpack_hbm, out_hbm,
               xin, acc, outbuf, coloff_v, stgi, stgf,
               coloff_s, cinv_s, segpack_s, semx, semp, semo):
    w = _wid()
    blk0 = w * RPW

    pltpu.sync_copy(coloff_hbm, coloff_v)
    _to_smem(coloff_v, coloff_s, W)
    cc96 = coloff_s[W - 1] + CP
    pltpu.sync_copy(cinv_hbm, stgf)
    _to_smem(stgf, cinv_s, W)
    pltpu.sync_copy(rowid_hbm, stgi)
    rid0 = _sload(stgi, blk0)
    rid1 = _sload(stgi, blk0 + RPW - 1)
    pltpu.sync_copy(segpack_hbm, stgi)
    _to_smem(stgi, segpack_s, H)

    zv = jnp.zeros((16,), jnp.float32)

    def obw(i, _):
        pltpu.make_async_copy(outbuf, out_hbm.at[0, i], semo).wait()
        return 0

    def segbody(r, carry):
        plo, phi = carry
        sv = segpack_s[r]
        i0 = jnp.bitwise_and(sv, 1023)
        n = lax.shift_right_logical(sv, 10)
        rv = (jnp.full((16,), 1.0, jnp.float32)
              / jnp.full((16,), n.astype(jnp.float32)))[0]

        def zbody(z, _):
            for u in range(4):
                acc[pl.ds(z * 64 + u * 16, 16)] = zv
            return 0

        lax.fori_loop(0, (cc96 + 63) // 64, zbody, 0)

        pltpu.async_copy(x_hbm.at[0, i0, pl.ds(0, WCH), :], xin.at[0], semp)

        def rowb(i, _):
            hnd = [None] * XCH
            hnd[0] = pltpu.make_async_copy(
                x_hbm.at[0, i, pl.ds(0, WCH), :], xin.at[0], semp)
            for ch in range(XCH):
                if ch + 1 < XCH:
                    hnd[ch + 1] = pltpu.async_copy(
                        x_hbm.at[0, i, pl.ds((ch + 1) * WCH, WCH), :],
                        xin.at[(ch + 1) % 2], semx)
                else:
                    @pl.when(i + 1 < i0 + n)
                    def _():
                        pltpu.async_copy(
                            x_hbm.at[0, i + 1, pl.ds(0, WCH), :],
                            xin.at[0], semp)
                hnd[ch].wait()
                xb = ch % 2
                cbase = ch * WCH

                def jbody(jh, _):
                    j = jh * 2
                    offa = coloff_s[cbase + j]
                    offb = coloff_s[cbase + j + 1]
                    qa = jnp.full((16,), rv * cinv_s[cbase + j], jnp.float32)
                    qb = jnp.full((16,), rv * cinv_s[cbase + j + 1], jnp.float32)
                    va = tuple(
                        xin[xb, j, pl.ds(v * 16, 16)] * qa for v in range(NV))
                    vb = tuple(
                        xin[xb, j + 1, pl.ds(v * 16, 16)] * qb
                        for v in range(NV))
                    for v in range(NV):
                        plsc.addupdate(acc.at[pl.ds(offa + v * 16, 16)], va[v])
                    for v in range(NV):
                        plsc.addupdate(acc.at[pl.ds(offb + v * 16, 16)], vb[v])
                    return 0

                lax.fori_loop(0, WCH // 2, jbody, 0)
            return 0

        lax.fori_loop(i0, i0 + n, rowb, 0)

        lax.fori_loop(plo, phi, obw, 0)

        def ebody(g, _):
            cv = coloff_v[pl.ds(g * 16, 16)]

            def cbody(c, _):
                c2 = c * 2
                v0 = plsc.load_gather(acc, [cv + c2])
                v1 = plsc.load_gather(acc, [cv + (c2 + 1)])
                outbuf[c2, pl.ds(g * 16, 16)] = v0
                outbuf[c2 + 1, pl.ds(g * 16, 16)] = v1
                return 0

            lax.fori_loop(0, C // 2, cbody, 0)
            return 0

        lax.fori_loop(0, W // 16, ebody, 0)

        lo = jnp.maximum(i0, blk0)
        hi = jnp.minimum(i0 + n, blk0 + RPW)

        def ob(i, _):
            pltpu.async_copy(outbuf, out_hbm.at[0, i], semo)
            return 0

        lax.fori_loop(lo, hi, ob, 0)
        return (lo, hi)

    flo, fhi = lax.fori_loop(rid0, rid1 + 1, segbody,
                             (jnp.int32(0), jnp.int32(0)))
    lax.fori_loop(flo, fhi, obw, 0)


_main = functools.partial(
    pl.kernel,
    out_type=jax.ShapeDtypeStruct((1, H, C, W), jnp.float32),
    mesh=_mesh,
    compiler_params=pltpu.CompilerParams(needs_layout_passes=False),
    scratch_types=[
        pltpu.VMEM((2, WCH, C), jnp.float32),
        pltpu.VMEM((ROWCAP,), jnp.float32),
        pltpu.VMEM((C, W), jnp.float32),
        pltpu.VMEM((W + 16,), jnp.int32),
        pltpu.VMEM((W + 16,), jnp.int32),
        pltpu.VMEM((W + 16,), jnp.float32),
        pltpu.SMEM((W,), jnp.int32),
        pltpu.SMEM((W,), jnp.float32),
        pltpu.SMEM((H,), jnp.int32),
        pltpu.SemaphoreType.DMA,
        pltpu.SemaphoreType.DMA,
        pltpu.SemaphoreType.DMA,
    ],
)(_main_body)


def kernel(input, h_mask, v_mask):
    hm = h_mask.astype(jnp.int32)
    vm = v_mask.astype(jnp.int32)
    rowid, colid96, cinv, segpack = _stage0(
        hm, hm.reshape(H, 1), vm, vm.reshape(W, 1))
    pad = lambda a: jnp.pad(a, (0, 16))
    out_t = _main(input, pad(colid96.reshape(W)), pad(cinv.reshape(W)),
                  pad(rowid.reshape(H)), pad(segpack.reshape(H)))
    return jnp.transpose(out_t, (0, 1, 3, 2))


# speedup vs baseline: 1.0013x; 1.0013x over previous
import functools

import jax
import jax.numpy as jnp
from jax import lax
from jax.experimental import pallas as pl
from jax.experimental.pallas import tpu as pltpu
from jax.experimental.pallas import tpu_sc as plsc

H = 512
W = 512
C = 96
NV = C // 16
SMAXP = 288
CP = C + 1
ROWCAP = SMAXP * CP
NC, NS = 2, 16
NW = NC * NS
RPW = H // NW
XCH = 4

_mesh = plsc.VectorSubcoreMesh(
    core_axis_name="c", subcore_axis_name="s", num_cores=NC, num_subcores=NS
)


def _wid():
    return lax.axis_index("s") * NC + lax.axis_index("c")


def _sload(ref, i):
    return plsc.load_gather(ref, [jnp.full((16,), i, jnp.int32)])[0]


def _to_smem(vmem_ref, smem_ref, n):
    def gbody(g, _):
        v = vmem_ref[pl.ds(g * 16, 16)]
        for l in range(16):
            smem_ref[g * 16 + l] = v[l]
        return 0

    lax.fori_loop(0, n // 16, gbody, 0)


def _stage0_body(hm_r_ref, hm_c_ref, vm_r_ref, vm_c_ref,
                 rowid_ref, colid96_ref, cinv_ref, segpack_ref):
    ii_r = lax.broadcasted_iota(jnp.int32, (1, W), 1)
    ii_c = lax.broadcasted_iota(jnp.int32, (H, 1), 0)
    i0m = lax.broadcasted_iota(jnp.int32, (H, W), 0)
    i1m = lax.broadcasted_iota(jnp.int32, (H, W), 1)
    le = (i0m <= i1m).astype(jnp.float32)
    ge = (i0m >= i1m).astype(jnp.float32)

    def rise_row(m):
        prev = jnp.concatenate([jnp.zeros((1, 1), jnp.int32), m[:, : W - 1]], axis=1)
        return jnp.where((m - prev > 0) & (ii_r > 0), 1.0, 0.0).astype(jnp.float32)

    def rise_col(m):
        prev = jnp.concatenate([jnp.zeros((1, 1), jnp.int32), m[: H - 1, :]], axis=0)
        return jnp.where((m - prev > 0) & (ii_c > 0), 1.0, 0.0).astype(jnp.float32)

    hrise_r = rise_row(hm_r_ref[...])
    hrise_c = rise_col(hm_c_ref[...])
    vrise_r = rise_row(vm_r_ref[...])
    vrise_c = rise_col(vm_c_ref[...])

    rowid_r = jnp.dot(hrise_r, le, preferred_element_type=jnp.float32)
    rowid_c = jnp.dot(ge, hrise_c, preferred_element_type=jnp.float32)
    colid_r = jnp.dot(vrise_r, le, preferred_element_type=jnp.float32)
    colid_c = jnp.dot(ge, vrise_c, preferred_element_type=jnp.float32)

    ecol = (colid_c == colid_r).astype(jnp.float32)
    ncol = jnp.sum(ecol, axis=0, keepdims=True)

    cmp = (i0m == rowid_r.astype(jnp.int32)).astype(jnp.float32)
    is_start = jnp.where(ii_r == 0, 1.0, hrise_r)
    i_f = ii_r.astype(jnp.float32)
    segn = jnp.sum(cmp, axis=1, keepdims=True)
    segstart = jnp.sum(cmp * is_start * i_f, axis=1, keepdims=True)

    rowid_ref[...] = rowid_c.astype(jnp.int32)
    colid96_ref[...] = (colid_r * 97.0).astype(jnp.int32)
    cinv_ref[...] = 1.0 / ncol
    segpack_ref[...] = (segstart + segn * 1024.0).astype(jnp.int32)


_stage0 = pl.pallas_call(
    _stage0_body,
    out_shape=[
        jax.ShapeDtypeStruct((H, 1), jnp.int32),
        jax.ShapeDtypeStruct((1, W), jnp.int32),
        jax.ShapeDtypeStruct((1, W), jnp.float32),
        jax.ShapeDtypeStruct((H, 1), jnp.int32),
    ],
)


WCH = W // XCH


def _main_body(x_hbm, coloff_hbm, cinv_hbm, rowid_hbm, segpack_hbm, out_hbm,
               xin, acc, outbuf, coloff_v, stgi, stgf,
               coloff_s, cinv_s, segpack_s, semx, semp, semo):
    w = _wid()
    blk0 = w * RPW

    pltpu.sync_copy(coloff_hbm, coloff_v)
    _to_smem(coloff_v, coloff_s, W)
    cc96 = coloff_s[W - 1] + CP
    pltpu.sync_copy(cinv_hbm, stgf)
    _to_smem(stgf, cinv_s, W)
    pltpu.sync_copy(rowid_hbm, stgi)
    rid0 = _sload(stgi, blk0)
    rid1 = _sload(stgi, blk0 + RPW - 1)
    pltpu.sync_copy(segpack_hbm, stgi)
    _to_smem(stgi, segpack_s, H)

    zv = jnp.zeros((16,), jnp.float32)

    def obw(i, _):
        pltpu.make_async_copy(outbuf, out_hbm.at[0, i], semo).wait()
        return 0

    def segbody(r, carry):
        plo, phi = carry
        sv = segpack_s[r]
        i0 = jnp.bitwise_and(sv, 1023)
        n = lax.shift_right_logical(sv, 10)
        rv = (jnp.full((16,), 1.0, jnp.float32)
              / jnp.full((16,), n.astype(jnp.float32)))[0]

        def zbody(z, _):
            for u in range(4):
                acc[pl.ds(z * 64 + u * 16, 16)] = zv
            return 0

        lax.fori_loop(0, (cc96 + 63) // 64, zbody, 0)

        pltpu.async_copy(x_hbm.at[0, i0, pl.ds(0, WCH), :], xin.at[0], semp)

        def rowb(i, _):
            hnd = [None] * XCH
            hnd[0] = pltpu.make_async_copy(
                x_hbm.at[0, i, pl.ds(0, WCH), :], xin.at[0], semp)
            for ch in range(XCH):
                if ch + 1 < XCH:
                    hnd[ch + 1] = pltpu.async_copy(
                        x_hbm.at[0, i, pl.ds((ch + 1) * WCH, WCH), :],
                        xin.at[(ch + 1) % 2], semx)
                else:
                    @pl.when(i + 1 < i0 + n)
                    def _():
                        pltpu.async_copy(
                            x_hbm.at[0, i + 1, pl.ds(0, WCH), :],
                            xin.at[0], semp)
                hnd[ch].wait()
                xb = ch % 2
                cbase = ch * WCH

                def jbody(jh, _):
                    j = jh * 2
                    offa = coloff_s[cbase + j]
                    offb = coloff_s[cbase + j + 1]
                    qa = jnp.full((16,), rv * cinv_s[cbase + j], jnp.float32)
                    qb = jnp.full((16,), rv * cinv_s[cbase + j + 1], jnp.float32)
                    va = tuple(
                        xin[xb, j, pl.ds(v * 16, 16)] * qa for v in range(NV))
                    vb = tuple(
                        xin[xb, j + 1, pl.ds(v * 16, 16)] * qb
                        for v in range(NV))
                    for v in range(NV):
                        plsc.addupdate(acc.at[pl.ds(offa + v * 16, 16)], va[v])
                    for v in range(NV):
                        plsc.addupdate(acc.at[pl.ds(offb + v * 16, 16)], vb[v])
                    return 0

                lax.fori_loop(0, WCH // 2, jbody, 0)
            return 0

        lax.fori_loop(i0, i0 + n, rowb, 0)

        lax.fori_loop(plo, phi, obw, 0)

        def ebody(g, _):
            cv = coloff_v[pl.ds(g * 16, 16)]

            def cbody(c, _):
                c2 = c * 2
                v0 = plsc.load_gather(acc, [cv + c2])
                v1 = plsc.load_gather(acc, [cv + (c2 + 1)])
                outbuf[c2, pl.ds(g * 16, 16)] = v0
                outbuf[c2 + 1, pl.ds(g * 16, 16)] = v1
                return 0

            lax.fori_loop(0, C // 2, cbody, 0)
            return 0

        lax.fori_loop(0, W // 16, ebody, 0)

        lo = jnp.maximum(i0, blk0)
        hi = jnp.minimum(i0 + n, blk0 + RPW)

        def ob(i, _):
            pltpu.async_copy(outbuf, out_hbm.at[0, i], semo)
            return 0

        lax.fori_loop(lo, hi, ob, 0)
        return (lo, hi)

    flo, fhi = lax.fori_loop(rid0, rid1 + 1, segbody,
                             (jnp.int32(0), jnp.int32(0)))
    lax.fori_loop(flo, fhi, obw, 0)


_main = functools.partial(
    pl.kernel,
    out_type=jax.ShapeDtypeStruct((1, H, C, W), jnp.float32),
    mesh=_mesh,
    compiler_params=pltpu.CompilerParams(needs_layout_passes=False),
    scratch_types=[
        pltpu.VMEM((2, WCH, C), jnp.float32),
        pltpu.VMEM((ROWCAP,), jnp.float32),
        pltpu.VMEM((C, W), jnp.float32),
        pltpu.VMEM((W + 16,), jnp.int32),
        pltpu.VMEM((W + 16,), jnp.int32),
        pltpu.VMEM((W + 16,), jnp.float32),
        pltpu.SMEM((W,), jnp.int32),
        pltpu.SMEM((W,), jnp.float32),
        pltpu.SMEM((H,), jnp.int32),
        pltpu.SemaphoreType.DMA,
        pltpu.SemaphoreType.DMA,
        pltpu.SemaphoreType.DMA,
    ],
)(_main_body)


def kernel(input, h_mask, v_mask):
    hm = h_mask.astype(jnp.int32)
    vm = v_mask.astype(jnp.int32)
    rowid, colid96, cinv, segpack = _stage0(
        hm, hm.reshape(H, 1), vm, vm.reshape(W, 1))
    pad = lambda a: jnp.pad(a, (0, 16))
    out_t = _main(input, pad(colid96.reshape(W)), pad(cinv.reshape(W)),
                  pad(rowid.reshape(H)), pad(segpack.reshape(H)))
    return jnp.transpose(out_t, (0, 1, 3, 2))


# speedup vs baseline: 1.0300x; 1.0287x over previous
import functools

import jax
import jax.numpy as jnp
from jax import lax
from jax.experimental import pallas as pl
from jax.experimental.pallas import tpu as pltpu
from jax.experimental.pallas import tpu_sc as plsc

H = 512
W = 512
C = 96
NV = C // 16
SMAXP = 288
CP = C + 1
ROWCAP = SMAXP * CP
NC, NS = 2, 16
NW = NC * NS
RPW = H // NW
XCH = 4

_mesh = plsc.VectorSubcoreMesh(
    core_axis_name="c", subcore_axis_name="s", num_cores=NC, num_subcores=NS
)


def _wid():
    return lax.axis_index("s") * NC + lax.axis_index("c")


def _sload(ref, i):
    return plsc.load_gather(ref, [jnp.full((16,), i, jnp.int32)])[0]


def _to_smem(vmem_ref, smem_ref, n):
    def gbody(g, _):
        v = vmem_ref[pl.ds(g * 16, 16)]
        for l in range(16):
            smem_ref[g * 16 + l] = v[l]
        return 0

    lax.fori_loop(0, n // 16, gbody, 0)


def _stage0_body(hm_r_ref, hm_c_ref, vm_r_ref, vm_c_ref,
                 rowid_ref, colid96_ref, cinv_ref, segpack_ref):
    ii_r = lax.broadcasted_iota(jnp.int32, (1, W), 1)
    ii_c = lax.broadcasted_iota(jnp.int32, (H, 1), 0)
    i0m = lax.broadcasted_iota(jnp.int32, (H, W), 0)
    i1m = lax.broadcasted_iota(jnp.int32, (H, W), 1)
    le = (i0m <= i1m).astype(jnp.float32)
    ge = (i0m >= i1m).astype(jnp.float32)

    def rise_row(m):
        prev = jnp.concatenate([jnp.zeros((1, 1), jnp.int32), m[:, : W - 1]], axis=1)
        return jnp.where((m - prev > 0) & (ii_r > 0), 1.0, 0.0).astype(jnp.float32)

    def rise_col(m):
        prev = jnp.concatenate([jnp.zeros((1, 1), jnp.int32), m[: H - 1, :]], axis=0)
        return jnp.where((m - prev > 0) & (ii_c > 0), 1.0, 0.0).astype(jnp.float32)

    hrise_r = rise_row(hm_r_ref[...])
    hrise_c = rise_col(hm_c_ref[...])
    vrise_r = rise_row(vm_r_ref[...])
    vrise_c = rise_col(vm_c_ref[...])

    rowid_r = jnp.dot(hrise_r, le, preferred_element_type=jnp.float32)
    rowid_c = jnp.dot(ge, hrise_c, preferred_element_type=jnp.float32)
    colid_r = jnp.dot(vrise_r, le, preferred_element_type=jnp.float32)
    colid_c = jnp.dot(ge, vrise_c, preferred_element_type=jnp.float32)

    ecol = (colid_c == colid_r).astype(jnp.float32)
    ncol = jnp.sum(ecol, axis=0, keepdims=True)

    cmp = (i0m == rowid_r.astype(jnp.int32)).astype(jnp.float32)
    is_start = jnp.where(ii_r == 0, 1.0, hrise_r)
    i_f = ii_r.astype(jnp.float32)
    segn = jnp.sum(cmp, axis=1, keepdims=True)
    segstart = jnp.sum(cmp * is_start * i_f, axis=1, keepdims=True)

    rowid_ref[...] = rowid_c.astype(jnp.int32)
    colid96_ref[...] = (colid_r * 97.0).astype(jnp.int32)
    cinv_ref[...] = 1.0 / ncol
    segpack_ref[...] = (segstart + segn * 1024.0).astype(jnp.int32)


_stage0 = pl.pallas_call(
    _stage0_body,
    out_shape=[
        jax.ShapeDtypeStruct((H, 1), jnp.int32),
        jax.ShapeDtypeStruct((1, W), jnp.int32),
        jax.ShapeDtypeStruct((1, W), jnp.float32),
        jax.ShapeDtypeStruct((H, 1), jnp.int32),
    ],
)


WCH = W // XCH


def _main_body(x_hbm, coloff_hbm, cinv_hbm, rowid_hbm, segpack_hbm, out_hbm,
               xin, acc, outbuf, coloff_v, stgi, stgf,
               coloff_s, cinv_s, segpack_s, semx, semp, semo):
    w = _wid()
    blk0 = w * RPW

    pltpu.sync_copy(coloff_hbm, coloff_v)
    _to_smem(coloff_v, coloff_s, W)
    cc96 = coloff_s[W - 1] + CP
    pltpu.sync_copy(cinv_hbm, stgf)
    _to_smem(stgf, cinv_s, W)
    pltpu.sync_copy(rowid_hbm, stgi)
    rid0 = _sload(stgi, blk0)
    rid1 = _sload(stgi, blk0 + RPW - 1)
    pltpu.sync_copy(segpack_hbm, stgi)
    _to_smem(stgi, segpack_s, H)

    zv = jnp.zeros((16,), jnp.float32)

    def obw(i, _):
        pltpu.make_async_copy(outbuf, out_hbm.at[0, i], semo).wait()
        return 0

    def segbody(r, carry):
        plo, phi = carry
        sv = segpack_s[r]
        i0 = jnp.bitwise_and(sv, 1023)
        n = lax.shift_right_logical(sv, 10)
        rv = (jnp.full((16,), 1.0, jnp.float32)
              / jnp.full((16,), n.astype(jnp.float32)))[0]

        pltpu.async_copy(x_hbm.at[0, i0, pl.ds(0, WCH), :], xin.at[0], semp)

        def zbody(z, _):
            for u in range(4):
                acc[pl.ds(z * 64 + u * 16, 16)] = zv
            return 0

        lax.fori_loop(0, (cc96 + 63) // 64, zbody, 0)

        def rowb(i, _):
            hnd = [None] * XCH
            hnd[0] = pltpu.make_async_copy(
                x_hbm.at[0, i, pl.ds(0, WCH), :], xin.at[0], semp)
            for ch in range(XCH):
                if ch + 1 < XCH:
                    hnd[ch + 1] = pltpu.async_copy(
                        x_hbm.at[0, i, pl.ds((ch + 1) * WCH, WCH), :],
                        xin.at[(ch + 1) % 2], semx)
                else:
                    @pl.when(i + 1 < i0 + n)
                    def _():
                        pltpu.async_copy(
                            x_hbm.at[0, i + 1, pl.ds(0, WCH), :],
                            xin.at[0], semp)
                hnd[ch].wait()
                xb = ch % 2
                cbase = ch * WCH

                def jbody(jh, _):
                    j = jh * 2
                    offa = coloff_s[cbase + j]
                    offb = coloff_s[cbase + j + 1]
                    qa = jnp.full((16,), rv * cinv_s[cbase + j], jnp.float32)
                    qb = jnp.full((16,), rv * cinv_s[cbase + j + 1], jnp.float32)
                    va = tuple(
                        xin[xb, j, pl.ds(v * 16, 16)] * qa for v in range(NV))
                    vb = tuple(
                        xin[xb, j + 1, pl.ds(v * 16, 16)] * qb
                        for v in range(NV))
                    for v in range(NV):
                        plsc.addupdate(acc.at[pl.ds(offa + v * 16, 16)], va[v])
                    for v in range(NV):
                        plsc.addupdate(acc.at[pl.ds(offb + v * 16, 16)], vb[v])
                    return 0

                lax.fori_loop(0, WCH // 2, jbody, 0)
            return 0

        lax.fori_loop(i0, i0 + n, rowb, 0)

        lax.fori_loop(plo, phi, obw, 0)

        def ebody(g, _):
            cv = coloff_v[pl.ds(g * 16, 16)]

            def cbody(c, _):
                c2 = c * 2
                v0 = plsc.load_gather(acc, [cv + c2])
                v1 = plsc.load_gather(acc, [cv + (c2 + 1)])
                outbuf[c2, pl.ds(g * 16, 16)] = v0
                outbuf[c2 + 1, pl.ds(g * 16, 16)] = v1
                return 0

            lax.fori_loop(0, C // 2, cbody, 0)
            return 0

        lax.fori_loop(0, W // 16, ebody, 0)

        lo = jnp.maximum(i0, blk0)
        hi = jnp.minimum(i0 + n, blk0 + RPW)

        def ob(i, _):
            pltpu.async_copy(outbuf, out_hbm.at[0, i], semo)
            return 0

        lax.fori_loop(lo, hi, ob, 0)
        return (lo, hi)

    flo, fhi = lax.fori_loop(rid0, rid1 + 1, segbody,
                             (jnp.int32(0), jnp.int32(0)))
    lax.fori_loop(flo, fhi, obw, 0)


_main = functools.partial(
    pl.kernel,
    out_type=jax.ShapeDtypeStruct((1, H, C, W), jnp.float32),
    mesh=_mesh,
    compiler_params=pltpu.CompilerParams(needs_layout_passes=False),
    scratch_types=[
        pltpu.VMEM((2, WCH, C), jnp.float32),
        pltpu.VMEM((ROWCAP,), jnp.float32),
        pltpu.VMEM((C, W), jnp.float32),
        pltpu.VMEM((W + 16,), jnp.int32),
        pltpu.VMEM((W + 16,), jnp.int32),
        pltpu.VMEM((W + 16,), jnp.float32),
        pltpu.SMEM((W,), jnp.int32),
        pltpu.SMEM((W,), jnp.float32),
        pltpu.SMEM((H,), jnp.int32),
        pltpu.SemaphoreType.DMA,
        pltpu.SemaphoreType.DMA,
        pltpu.SemaphoreType.DMA,
    ],
)(_main_body)


def kernel(input, h_mask, v_mask):
    hm = h_mask.astype(jnp.int32)
    vm = v_mask.astype(jnp.int32)
    rowid, colid96, cinv, segpack = _stage0(
        hm, hm.reshape(H, 1), vm, vm.reshape(W, 1))
    pad = lambda a: jnp.pad(a, (0, 16))
    out_t = _main(input, pad(colid96.reshape(W)), pad(cinv.reshape(W)),
                  pad(rowid.reshape(H)), pad(segpack.reshape(H)))
    return jnp.transpose(out_t, (0, 1, 3, 2))
